# baseline (speedup 1.0000x reference)
import jax
import jax.numpy as jnp
from jax import lax
from jax.experimental import pallas as pl
from jax.experimental.pallas import tpu as pltpu
from jax.experimental.pallas import tpu_sc as plsc

N = 100000
E = 1600000
RNN = 20
RNN_E = 8
HID = 64
LAT = 32

CH = 128
GK = 8
E_PAD = 1605632
NCH = E_PAD // CH
G_PAD = 3211264
DUMMY = N
N_ACC = 100096

_BE1 = 12800
_BE3 = 6400
_BN = 4352


def _leaky(v):
    return jnp.where(v >= 0, v, 0.01 * v)


def _dot(a, b):
    return jnp.dot(a, b, preferred_element_type=jnp.float32)


def _k1_body(ea_ref, hh_ref, hc_ref, wi_ref, wh_ref, b_ref,
             h2_ref, c2_ref, hp_ref):
    g = _dot(wi_ref[...], ea_ref[...]) + _dot(wh_ref[...], hh_ref[...]) \
        + b_ref[...]
    i = jax.nn.sigmoid(g[:RNN_E])
    f = jax.nn.sigmoid(g[RNN_E:2 * RNN_E])
    gg = jnp.tanh(g[2 * RNN_E:3 * RNN_E])
    o = jax.nn.sigmoid(g[3 * RNN_E:])
    c2 = f * hc_ref[...] + i * gg
    h2 = o * jnp.tanh(c2)
    h2_ref[...] = h2
    c2_ref[...] = c2
    h2t = jnp.transpose(h2)
    hp_ref[:, :16] = jnp.concatenate(
        [h2t, jnp.zeros((h2t.shape[0], 16 - RNN_E), jnp.float32)], axis=1)


def _k1(ea_t, heh_t, hec_t, wi, wh, b):
    return pl.pallas_call(
        _k1_body,
        grid=(E // _BE1,),
        in_specs=[
            pl.BlockSpec((4, _BE1), lambda i: (0, i)),
            pl.BlockSpec((RNN_E, _BE1), lambda i: (0, i)),
            pl.BlockSpec((RNN_E, _BE1), lambda i: (0, i)),
            pl.BlockSpec((4 * RNN_E, 4), lambda i: (0, 0)),
            pl.BlockSpec((4 * RNN_E, RNN_E), lambda i: (0, 0)),
            pl.BlockSpec((4 * RNN_E, 1), lambda i: (0, 0)),
        ],
        out_specs=[
            pl.BlockSpec((RNN_E, _BE1), lambda i: (0, i)),
            pl.BlockSpec((RNN_E, _BE1), lambda i: (0, i)),
            pl.BlockSpec((_BE1, 128), lambda i: (i, 0)),
        ],
        out_shape=[
            jax.ShapeDtypeStruct((RNN_E, E), jnp.float32),
            jax.ShapeDtypeStruct((RNN_E, E), jnp.float32),
            jax.ShapeDtypeStruct((E_PAD, 128), jnp.float32),
        ],
    )(ea_t, heh_t, hec_t, wi, wh, b)


def _s1_body(vals_hbm, idx_hbm, out_hbm, idxb, valb, acc, sem):
    del sem
    cid = lax.axis_index("c")
    sid = lax.axis_index("s")

    @pl.loop(0, GK * CH)
    def _zero(i):
        valb[i] = jnp.zeros((16,), jnp.float32)

    zb = sid * 6256
    for t in range(6):
        pltpu.sync_copy(valb, acc.at[pl.ds(zb + t * 1024, 1024)])
    pltpu.sync_copy(valb.at[pl.ds(0, 112)], acc.at[pl.ds(zb + 6144, 112)])
    plsc.subcore_barrier()

    base_chunk = cid * (NCH // 2) + sid * (NCH // 32)

    @pl.loop(0, NCH // 32 // GK)
    def _grp(g):
        cb = base_chunk + g * GK
        pltpu.sync_copy(idx_hbm.at[pl.ds(cb, GK)], idxb)
        pltpu.sync_copy(vals_hbm.at[pl.ds(cb * CH, GK * CH), pl.ds(0, 16)],
                        valb)
        for k in range(GK):
            pltpu.sync_copy(valb.at[pl.ds(k * CH, CH)],
                            acc.at[idxb.at[k]], add=True)

    plsc.subcore_barrier()
    pltpu.sync_copy(acc.at[pl.ds(sid * 6256, 6256)],
                    out_hbm.at[cid, pl.ds(sid * 6256, 6256)])


def _s1(vals, idx2):
    kfn = pl.kernel(
        _s1_body,
        out_type=jax.ShapeDtypeStruct((2, N_ACC, 16), jnp.float32),
        mesh=plsc.VectorSubcoreMesh(core_axis_name="c", subcore_axis_name="s"),
        scratch_types=[
            pltpu.VMEM((GK, CH), jnp.int32),
            pltpu.VMEM((GK * CH, 16), jnp.float32),
            pltpu.VMEM_SHARED((N_ACC, 16), jnp.float32),
            pltpu.SemaphoreType.DMA,
        ],
        compiler_params=pltpu.CompilerParams(use_tc_tiling_on_sc=False),
    )
    return kfn(vals, idx2)


def _k2_body(x_ref, hh_ref, hc_ref, p_ref,
             wii_ref, wif_ref, wig_ref, wio_ref,
             whi_ref, whf_ref, whg_ref, who_ref,
             bi_ref, bf_ref, bg_ref, bo_ref,
             h2_ref, c2_ref, xcp_ref):
    x5 = x_ref[...][:5]
    hh = hh_ref[...]
    i = jax.nn.sigmoid(_dot(wii_ref[...], x5) + _dot(whi_ref[...], hh)
                       + bi_ref[...])
    f = jax.nn.sigmoid(_dot(wif_ref[...], x5) + _dot(whf_ref[...], hh)
                       + bf_ref[...])
    gg = jnp.tanh(_dot(wig_ref[...], x5) + _dot(whg_ref[...], hh)
                  + bg_ref[...])
    o = jax.nn.sigmoid(_dot(wio_ref[...], x5) + _dot(who_ref[...], hh)
                       + bo_ref[...])
    c2 = f * hc_ref[...] + i * gg
    h2 = o * jnp.tanh(c2)
    h2_ref[...] = h2
    c2_ref[...] = c2
    en = p_ref[0] + p_ref[1]
    h2t = jnp.transpose(h2)
    xcp_ref[...] = jnp.concatenate(
        [h2t, en[:, :RNN_E], jnp.zeros((h2t.shape[0], 4), jnp.float32)],
        axis=1)


def _k2(x_t, hnh_t, hnc_t, p, wi4, wh4, b4):
    small = ([pl.BlockSpec((RNN, 5), lambda i: (0, 0))] * 4
             + [pl.BlockSpec((RNN, RNN), lambda i: (0, 0))] * 4
             + [pl.BlockSpec((RNN, 1), lambda i: (0, 0))] * 4)
    return pl.pallas_call(
        _k2_body,
        grid=(N_ACC // _BN,),
        in_specs=[
            pl.BlockSpec((10, _BN), lambda i: (0, i)),
            pl.BlockSpec((RNN, _BN), lambda i: (0, i)),
            pl.BlockSpec((RNN, _BN), lambda i: (0, i)),
            pl.BlockSpec((2, _BN, 16), lambda i: (0, i, 0)),
        ] + small,
        out_specs=[
            pl.BlockSpec((RNN, _BN), lambda i: (0, i)),
            pl.BlockSpec((RNN, _BN), lambda i: (0, i)),
            pl.BlockSpec((_BN, 32), lambda i: (i, 0)),
        ],
        out_shape=[
            jax.ShapeDtypeStruct((RNN, N_ACC), jnp.float32),
            jax.ShapeDtypeStruct((RNN, N_ACC), jnp.float32),
            jax.ShapeDtypeStruct((N_ACC, 32), jnp.float32),
        ],
    )(x_t, hnh_t, hnc_t, p, *wi4, *wh4, *b4)


def _sg_body(table_hbm, idx_hbm, out_hbm, idxb, rows, sem):
    wid = lax.axis_index("c") * 16 + lax.axis_index("s")
    base = wid * (G_PAD // 32)

    @pl.loop(0, G_PAD // 32 // (GK * CH))
    def _grp(g):
        b = base + g * GK * CH
        pltpu.sync_copy(idx_hbm.at[pl.ds(b, GK * CH)], idxb)
        cps = [pltpu.async_copy(table_hbm.at[idxb.at[pl.ds(k * CH, CH)]],
                                rows.at[pl.ds(k * CH, CH)], sem)
               for k in range(GK)]
        for cp in cps:
            cp.wait()
        pltpu.sync_copy(rows, out_hbm.at[pl.ds(b, GK * CH)])


def _sg(table, idxg):
    kfn = pl.kernel(
        _sg_body,
        out_type=jax.ShapeDtypeStruct((G_PAD, 32), jnp.float32),
        mesh=plsc.VectorSubcoreMesh(core_axis_name="c", subcore_axis_name="s"),
        scratch_types=[
            pltpu.VMEM((GK * CH,), jnp.int32),
            pltpu.VMEM((GK * CH, 32), jnp.float32),
            pltpu.SemaphoreType.DMA,
        ],
        compiler_params=pltpu.CompilerParams(use_tc_tiling_on_sc=False),
    )
    return kfn(table, idxg)


def _k3_body(gs_ref, gd_ref, ea_ref, w1a_ref, w1b_ref, w1c_ref, b1_ref,
             w2_ref, b2_ref, o_ref):
    pre = (_dot(gs_ref[...], w1a_ref[...])
           + _dot(gd_ref[...], w1b_ref[...])
           + _dot(ea_ref[...], w1c_ref[...])
           + b1_ref[...])
    eh = _leaky(pre)
    o_ref[...] = _dot(eh, w2_ref[...]) + b2_ref[...]


def _k3(g, eap, w1a, w1b, w1c, b1, w2, b2):
    nb = E // _BE3
    return pl.pallas_call(
        _k3_body,
        grid=(nb,),
        in_specs=[
            pl.BlockSpec((_BE3 // 4, 128), lambda i: (i, 0)),
            pl.BlockSpec((_BE3 // 4, 128), lambda i, _nb=nb: (i + _nb, 0)),
            pl.BlockSpec((_BE3 // 4, 128), lambda i: (i, 0)),
            pl.BlockSpec((128, 4 * HID), lambda i: (0, 0)),
            pl.BlockSpec((128, 4 * HID), lambda i: (0, 0)),
            pl.BlockSpec((128, 4 * HID), lambda i: (0, 0)),
            pl.BlockSpec((1, 4 * HID), lambda i: (0, 0)),
            pl.BlockSpec((4 * HID, 128), lambda i: (0, 0)),
            pl.BlockSpec((1, 128), lambda i: (0, 0)),
        ],
        out_specs=[pl.BlockSpec((_BE3 // 4, 128), lambda i: (i, 0))],
        out_shape=[jax.ShapeDtypeStruct((E_PAD // 4, 128), jnp.float32)],
    )(g, g, eap, w1a, w1b, w1c, b1, w2, b2)[0]


def _s2_body(vals_hbm, idx_hbm, out_hbm, idxb, valb, acc, sem):
    del sem
    cid = lax.axis_index("c")
    sid = lax.axis_index("s")

    @pl.loop(0, GK * CH)
    def _zero(i):
        valb[i] = jnp.zeros((16,), jnp.float32)

    zb = sid * 6256
    for t in range(6):
        pltpu.sync_copy(valb, acc.at[pl.ds(zb + t * 1024, 1024)])
    pltpu.sync_copy(valb.at[pl.ds(0, 112)], acc.at[pl.ds(zb + 6144, 112)])
    plsc.subcore_barrier()

    base_chunk = sid * (NCH // 16)

    @pl.loop(0, NCH // 16 // GK)
    def _grp(g):
        cb = base_chunk + g * GK
        pltpu.sync_copy(idx_hbm.at[pl.ds(cb, GK)], idxb)
        pltpu.sync_copy(vals_hbm.at[pl.ds(cb * CH, GK * CH),
                                    pl.ds(16 * cid, 16)], valb)
        for k in range(GK):
            pltpu.sync_copy(valb.at[pl.ds(k * CH, CH)],
                            acc.at[idxb.at[k]], add=True)

    plsc.subcore_barrier()
    pltpu.sync_copy(acc.at[pl.ds(sid * 6256, 6256)],
                    out_hbm.at[cid, pl.ds(sid * 6256, 6256)])


def _s2(vals, idx2):
    kfn = pl.kernel(
        _s2_body,
        out_type=jax.ShapeDtypeStruct((2, N_ACC, 16), jnp.float32),
        mesh=plsc.VectorSubcoreMesh(core_axis_name="c", subcore_axis_name="s"),
        scratch_types=[
            pltpu.VMEM((GK, CH), jnp.int32),
            pltpu.VMEM((GK * CH, 16), jnp.float32),
            pltpu.VMEM_SHARED((N_ACC, 16), jnp.float32),
            pltpu.SemaphoreType.DMA,
        ],
        compiler_params=pltpu.CompilerParams(use_tc_tiling_on_sc=False),
    )
    return kfn(vals, idx2)


def _k4_body(x_ref, xcp_ref, a_ref, w1a_ref, w1b_ref, w1c_ref, b1_ref,
             w2_ref, b2_ref, w3_ref, b3_ref, o_ref):
    h1 = _leaky(_dot(xcp_ref[...], w1a_ref[...])
                + _dot(a_ref[0], w1b_ref[...])
                + _dot(a_ref[1], w1c_ref[...])
                + b1_ref[...])
    h2 = _leaky(_dot(h1, w2_ref[...]) + b2_ref[...])
    no = _dot(h2, w3_ref[...]) + b3_ref[...]
    mask = jnp.any(x_ref[...][5:] != 0.0, axis=0, keepdims=True)
    o_ref[...] = jnp.where(mask, jnp.transpose(no), 0.0)


def _k4(x_t, xcp, agg, w1a, w1b, w1c, b1, w2, b2, w3, b3):
    return pl.pallas_call(
        _k4_body,
        grid=(N_ACC // _BN,),
        in_specs=[
            pl.BlockSpec((10, _BN), lambda i: (0, i)),
            pl.BlockSpec((_BN, 32), lambda i: (i, 0)),
            pl.BlockSpec((2, _BN, 16), lambda i: (0, i, 0)),
            pl.BlockSpec((32, HID), lambda i: (0, 0)),
            pl.BlockSpec((16, HID), lambda i: (0, 0)),
            pl.BlockSpec((16, HID), lambda i: (0, 0)),
            pl.BlockSpec((1, HID), lambda i: (0, 0)),
            pl.BlockSpec((HID, HID), lambda i: (0, 0)),
            pl.BlockSpec((1, HID), lambda i: (0, 0)),
            pl.BlockSpec((HID, 4), lambda i: (0, 0)),
            pl.BlockSpec((1, 4), lambda i: (0, 0)),
        ],
        out_specs=[pl.BlockSpec((4, _BN), lambda i: (0, i))],
        out_shape=[jax.ShapeDtypeStruct((4, N_ACC), jnp.float32)],
    )(x_t, xcp, agg, w1a, w1b, w1c, b1, w2, b2, w3, b3)[0]


def kernel(x, edge_index, edge_attr, h_node_h, h_node_c, h_edge_h, h_edge_c,
           Wih_n, Whh_n, bih_n, bhh_n, Wih_e, Whh_e, bih_e, bhh_e,
           We1, be1, We2, be2, Wc1, bc1, Wc2, bc2, Wc3, bc3):
    row = edge_index[0]
    col = edge_index[1]
    idx2 = jnp.pad(row, (0, E_PAD - E), constant_values=DUMMY).reshape(NCH, CH)
    idxg = jnp.pad(jnp.concatenate([row, col]), (0, G_PAD - 2 * E))

    ea_t = edge_attr.T
    b_e = (bih_e + bhh_e)[:, None]
    h_e2, c_e2, h2pad = _k1(ea_t, h_edge_h[0].T, h_edge_c[0].T,
                            Wih_e, Whh_e, b_e)

    p = _s1(h2pad, idx2)

    b_n = bih_n + bhh_n
    wi4 = [Wih_n[k * RNN:(k + 1) * RNN] for k in range(4)]
    wh4 = [Whh_n[k * RNN:(k + 1) * RNN] for k in range(4)]
    b4 = [b_n[k * RNN:(k + 1) * RNN][:, None] for k in range(4)]
    npad = ((0, 0), (0, N_ACC - N))
    x_t = jnp.pad(x.T, npad)
    h_n2, c_n2, xcp = _k2(x_t, jnp.pad(h_node_h[0].T, npad),
                          jnp.pad(h_node_c[0].T, npad), p, wi4, wh4, b4)

    g = _sg(xcp, idxg).reshape(G_PAD // 4, 128)

    eye4 = jnp.eye(4, dtype=jnp.float32)
    w1a = jnp.kron(eye4, jnp.pad(We1[:28], ((0, 4), (0, 0))))
    w1b = jnp.kron(eye4, jnp.pad(We1[28:56], ((0, 4), (0, 0))))
    w1c = jnp.kron(eye4, jnp.pad(We1[56:60], ((0, 28), (0, 0))))
    w2 = jnp.kron(eye4, We2)
    eap = jnp.pad(edge_attr, ((0, 0), (0, 28))).reshape(E // 4, 128)
    eo = _k3(g, eap, w1a, w1b, w1c, jnp.tile(be1, 4)[None, :],
             w2, jnp.tile(be2, 4)[None, :])

    agg = _s2(eo.reshape(E_PAD, 32), idx2)

    wc1a = jnp.pad(Wc1[:28], ((0, 4), (0, 0)))
    out_t = _k4(x_t, xcp, agg, wc1a, Wc1[28:44], Wc1[44:60], bc1[None, :],
                Wc2, bc2[None, :], Wc3, bc3[None, :])

    return (out_t[:, :N].T, h_n2[:, :N].T[None], c_n2[:, :N].T[None],
            h_e2.T[None], c_e2.T[None])


# speedup vs baseline: 1.1637x; 1.1637x over previous
import jax
import jax.numpy as jnp
from jax import lax
from jax.experimental import pallas as pl
from jax.experimental.pallas import tpu as pltpu
from jax.experimental.pallas import tpu_sc as plsc

N = 100000
E = 1600000
RNN = 20
RNN_E = 8
HID = 64
LAT = 32

CH = 128
GK = 8
E_PAD = 1605632
NCH = E_PAD // CH
G_PAD = 3211264
DUMMY = N
N_ACC = 100096

_BE1 = 12800
_BE3 = 6400
_BN = 4352


def _leaky(v):
    return jnp.where(v >= 0, v, 0.01 * v)


def _dot(a, b):
    return jnp.dot(a, b, preferred_element_type=jnp.float32)


def _k1_body(ea_ref, hh_ref, hc_ref, wi_ref, wh_ref, b_ref,
             h2_ref, c2_ref, hp_ref):
    g = _dot(wi_ref[...], ea_ref[...]) + _dot(wh_ref[...], hh_ref[...]) \
        + b_ref[...]
    i = jax.nn.sigmoid(g[:RNN_E])
    f = jax.nn.sigmoid(g[RNN_E:2 * RNN_E])
    gg = jnp.tanh(g[2 * RNN_E:3 * RNN_E])
    o = jax.nn.sigmoid(g[3 * RNN_E:])
    c2 = f * hc_ref[...] + i * gg
    h2 = o * jnp.tanh(c2)
    h2_ref[...] = h2
    c2_ref[...] = c2
    h2t = jnp.transpose(h2)
    hp_ref[:, :16] = jnp.concatenate(
        [h2t, jnp.zeros((h2t.shape[0], 16 - RNN_E), jnp.float32)], axis=1)


def _k1(ea_t, heh_t, hec_t, wi, wh, b):
    return pl.pallas_call(
        _k1_body,
        grid=(E // _BE1,),
        in_specs=[
            pl.BlockSpec((4, _BE1), lambda i: (0, i)),
            pl.BlockSpec((RNN_E, _BE1), lambda i: (0, i)),
            pl.BlockSpec((RNN_E, _BE1), lambda i: (0, i)),
            pl.BlockSpec((4 * RNN_E, 4), lambda i: (0, 0)),
            pl.BlockSpec((4 * RNN_E, RNN_E), lambda i: (0, 0)),
            pl.BlockSpec((4 * RNN_E, 1), lambda i: (0, 0)),
        ],
        out_specs=[
            pl.BlockSpec((RNN_E, _BE1), lambda i: (0, i)),
            pl.BlockSpec((RNN_E, _BE1), lambda i: (0, i)),
            pl.BlockSpec((_BE1, 128), lambda i: (i, 0)),
        ],
        out_shape=[
            jax.ShapeDtypeStruct((RNN_E, E), jnp.float32),
            jax.ShapeDtypeStruct((RNN_E, E), jnp.float32),
            jax.ShapeDtypeStruct((E_PAD, 128), jnp.float32),
        ],
    )(ea_t, heh_t, hec_t, wi, wh, b)


def _s1_body(vals_hbm, idx_hbm, out_hbm, idxb, valb, acc, sem):
    del sem
    cid = lax.axis_index("c")
    sid = lax.axis_index("s")

    @pl.loop(0, GK * CH)
    def _zero(i):
        valb[i] = jnp.zeros((16,), jnp.float32)

    zb = sid * 6256
    for t in range(6):
        pltpu.sync_copy(valb, acc.at[pl.ds(zb + t * 1024, 1024)])
    pltpu.sync_copy(valb.at[pl.ds(0, 112)], acc.at[pl.ds(zb + 6144, 112)])
    plsc.subcore_barrier()

    base_chunk = cid * (NCH // 2) + sid * (NCH // 32)

    @pl.loop(0, NCH // 32 // GK)
    def _grp(g):
        cb = base_chunk + g * GK
        pltpu.sync_copy(idx_hbm.at[pl.ds(cb, GK)], idxb)
        pltpu.sync_copy(vals_hbm.at[pl.ds(cb * CH, GK * CH), pl.ds(0, 16)],
                        valb)
        for k in range(GK):
            pltpu.sync_copy(valb.at[pl.ds(k * CH, CH)],
                            acc.at[idxb.at[k]], add=True)

    plsc.subcore_barrier()
    pltpu.sync_copy(acc.at[pl.ds(sid * 6256, 6256)],
                    out_hbm.at[cid, pl.ds(sid * 6256, 6256)])


def _s1(vals, idx2):
    kfn = pl.kernel(
        _s1_body,
        out_type=jax.ShapeDtypeStruct((2, N_ACC, 16), jnp.float32),
        mesh=plsc.VectorSubcoreMesh(core_axis_name="c", subcore_axis_name="s"),
        scratch_types=[
            pltpu.VMEM((GK, CH), jnp.int32),
            pltpu.VMEM((GK * CH, 16), jnp.float32),
            pltpu.VMEM_SHARED((N_ACC, 16), jnp.float32),
            pltpu.SemaphoreType.DMA,
        ],
        compiler_params=pltpu.CompilerParams(use_tc_tiling_on_sc=False),
    )
    return kfn(vals, idx2)


def _k2_body(x_ref, hh_ref, hc_ref, p_ref,
             wii_ref, wif_ref, wig_ref, wio_ref,
             whi_ref, whf_ref, whg_ref, who_ref,
             bi_ref, bf_ref, bg_ref, bo_ref,
             h2_ref, c2_ref, xcp_ref):
    x5 = x_ref[...][:5]
    hh = hh_ref[...]
    i = jax.nn.sigmoid(_dot(wii_ref[...], x5) + _dot(whi_ref[...], hh)
                       + bi_ref[...])
    f = jax.nn.sigmoid(_dot(wif_ref[...], x5) + _dot(whf_ref[...], hh)
                       + bf_ref[...])
    gg = jnp.tanh(_dot(wig_ref[...], x5) + _dot(whg_ref[...], hh)
                  + bg_ref[...])
    o = jax.nn.sigmoid(_dot(wio_ref[...], x5) + _dot(who_ref[...], hh)
                       + bo_ref[...])
    c2 = f * hc_ref[...] + i * gg
    h2 = o * jnp.tanh(c2)
    h2_ref[...] = h2
    c2_ref[...] = c2
    en = p_ref[0] + p_ref[1]
    h2t = jnp.transpose(h2)
    xcp_ref[...] = jnp.concatenate(
        [h2t, en[:, :RNN_E], jnp.zeros((h2t.shape[0], 4), jnp.float32)],
        axis=1)


def _k2(x_t, hnh_t, hnc_t, p, wi4, wh4, b4):
    small = ([pl.BlockSpec((RNN, 5), lambda i: (0, 0))] * 4
             + [pl.BlockSpec((RNN, RNN), lambda i: (0, 0))] * 4
             + [pl.BlockSpec((RNN, 1), lambda i: (0, 0))] * 4)
    return pl.pallas_call(
        _k2_body,
        grid=(N_ACC // _BN,),
        in_specs=[
            pl.BlockSpec((10, _BN), lambda i: (0, i)),
            pl.BlockSpec((RNN, _BN), lambda i: (0, i)),
            pl.BlockSpec((RNN, _BN), lambda i: (0, i)),
            pl.BlockSpec((2, _BN, 16), lambda i: (0, i, 0)),
        ] + small,
        out_specs=[
            pl.BlockSpec((RNN, _BN), lambda i: (0, i)),
            pl.BlockSpec((RNN, _BN), lambda i: (0, i)),
            pl.BlockSpec((_BN, 32), lambda i: (i, 0)),
        ],
        out_shape=[
            jax.ShapeDtypeStruct((RNN, N_ACC), jnp.float32),
            jax.ShapeDtypeStruct((RNN, N_ACC), jnp.float32),
            jax.ShapeDtypeStruct((N_ACC, 32), jnp.float32),
        ],
    )(x_t, hnh_t, hnc_t, p, *wi4, *wh4, *b4)


def _sg_body(table_hbm, idx_hbm, out_hbm, idxb, rows, sem):
    wid = lax.axis_index("c") * 16 + lax.axis_index("s")
    base = wid * (G_PAD // 32)

    @pl.loop(0, G_PAD // 32 // (GK * CH))
    def _grp(g):
        b = base + g * GK * CH
        pltpu.sync_copy(idx_hbm.at[pl.ds(b, GK * CH)], idxb)
        cps = [pltpu.async_copy(table_hbm.at[idxb.at[pl.ds(k * CH, CH)]],
                                rows.at[pl.ds(k * CH, CH)], sem)
               for k in range(GK)]
        for cp in cps:
            cp.wait()
        pltpu.sync_copy(rows, out_hbm.at[pl.ds(b, GK * CH)])


def _sg(table, idxg):
    kfn = pl.kernel(
        _sg_body,
        out_type=jax.ShapeDtypeStruct((G_PAD, 32), jnp.float32),
        mesh=plsc.VectorSubcoreMesh(core_axis_name="c", subcore_axis_name="s"),
        scratch_types=[
            pltpu.VMEM((GK * CH,), jnp.int32),
            pltpu.VMEM((GK * CH, 32), jnp.float32),
            pltpu.SemaphoreType.DMA,
        ],
        compiler_params=pltpu.CompilerParams(use_tc_tiling_on_sc=False),
    )
    return kfn(table, idxg)


def _k3_body(gs_ref, gd_ref, ea_ref, w1a_ref, w1b_ref, w1c_ref, b1_ref,
             w2_ref, b2_ref, o_ref):
    pre = (_dot(gs_ref[...], w1a_ref[...])
           + _dot(gd_ref[...], w1b_ref[...])
           + _dot(ea_ref[...], w1c_ref[...])
           + b1_ref[...])
    eh = _leaky(pre)
    o_ref[...] = _dot(eh, w2_ref[...]) + b2_ref[...]


def _k3(g, eap, w1a, w1b, w1c, b1, w2, b2):
    nb = E // _BE3
    return pl.pallas_call(
        _k3_body,
        grid=(nb,),
        in_specs=[
            pl.BlockSpec((_BE3 // 4, 128), lambda i: (i, 0)),
            pl.BlockSpec((_BE3 // 4, 128), lambda i, _nb=nb: (i + _nb, 0)),
            pl.BlockSpec((_BE3 // 4, 16), lambda i: (i, 0)),
            pl.BlockSpec((128, 4 * HID), lambda i: (0, 0)),
            pl.BlockSpec((128, 4 * HID), lambda i: (0, 0)),
            pl.BlockSpec((16, 4 * HID), lambda i: (0, 0)),
            pl.BlockSpec((1, 4 * HID), lambda i: (0, 0)),
            pl.BlockSpec((4 * HID, 128), lambda i: (0, 0)),
            pl.BlockSpec((1, 128), lambda i: (0, 0)),
        ],
        out_specs=[pl.BlockSpec((_BE3 // 4, 128), lambda i: (i, 0))],
        out_shape=[jax.ShapeDtypeStruct((E_PAD // 4, 128), jnp.float32)],
    )(g, g, eap, w1a, w1b, w1c, b1, w2, b2)[0]


def _s2_body(vals_hbm, idx_hbm, out_hbm, idxb, valb, acc, sem):
    del sem
    cid = lax.axis_index("c")
    sid = lax.axis_index("s")

    @pl.loop(0, GK * CH)
    def _zero(i):
        valb[i] = jnp.zeros((16,), jnp.float32)

    zb = sid * 6256
    for t in range(6):
        pltpu.sync_copy(valb, acc.at[pl.ds(zb + t * 1024, 1024)])
    pltpu.sync_copy(valb.at[pl.ds(0, 112)], acc.at[pl.ds(zb + 6144, 112)])
    plsc.subcore_barrier()

    base_chunk = sid * (NCH // 16)

    @pl.loop(0, NCH // 16 // GK)
    def _grp(g):
        cb = base_chunk + g * GK
        pltpu.sync_copy(idx_hbm.at[pl.ds(cb, GK)], idxb)
        pltpu.sync_copy(vals_hbm.at[pl.ds(cb * CH, GK * CH),
                                    pl.ds(16 * cid, 16)], valb)
        for k in range(GK):
            pltpu.sync_copy(valb.at[pl.ds(k * CH, CH)],
                            acc.at[idxb.at[k]], add=True)

    plsc.subcore_barrier()
    pltpu.sync_copy(acc.at[pl.ds(sid * 6256, 6256)],
                    out_hbm.at[cid, pl.ds(sid * 6256, 6256)])


def _s2(vals, idx2):
    kfn = pl.kernel(
        _s2_body,
        out_type=jax.ShapeDtypeStruct((2, N_ACC, 16), jnp.float32),
        mesh=plsc.VectorSubcoreMesh(core_axis_name="c", subcore_axis_name="s"),
        scratch_types=[
            pltpu.VMEM((GK, CH), jnp.int32),
            pltpu.VMEM((GK * CH, 16), jnp.float32),
            pltpu.VMEM_SHARED((N_ACC, 16), jnp.float32),
            pltpu.SemaphoreType.DMA,
        ],
        compiler_params=pltpu.CompilerParams(use_tc_tiling_on_sc=False),
    )
    return kfn(vals, idx2)


def _k4_body(x_ref, xcp_ref, a_ref, w1a_ref, w1b_ref, w1c_ref, b1_ref,
             w2_ref, b2_ref, w3_ref, b3_ref, o_ref):
    h1 = _leaky(_dot(xcp_ref[...], w1a_ref[...])
                + _dot(a_ref[0], w1b_ref[...])
                + _dot(a_ref[1], w1c_ref[...])
                + b1_ref[...])
    h2 = _leaky(_dot(h1, w2_ref[...]) + b2_ref[...])
    no = _dot(h2, w3_ref[...]) + b3_ref[...]
    mask = jnp.any(x_ref[...][5:] != 0.0, axis=0, keepdims=True)
    o_ref[...] = jnp.where(mask, jnp.transpose(no), 0.0)


def _k4(x_t, xcp, agg, w1a, w1b, w1c, b1, w2, b2, w3, b3):
    return pl.pallas_call(
        _k4_body,
        grid=(N_ACC // _BN,),
        in_specs=[
            pl.BlockSpec((10, _BN), lambda i: (0, i)),
            pl.BlockSpec((_BN, 32), lambda i: (i, 0)),
            pl.BlockSpec((2, _BN, 16), lambda i: (0, i, 0)),
            pl.BlockSpec((32, HID), lambda i: (0, 0)),
            pl.BlockSpec((16, HID), lambda i: (0, 0)),
            pl.BlockSpec((16, HID), lambda i: (0, 0)),
            pl.BlockSpec((1, HID), lambda i: (0, 0)),
            pl.BlockSpec((HID, HID), lambda i: (0, 0)),
            pl.BlockSpec((1, HID), lambda i: (0, 0)),
            pl.BlockSpec((HID, 4), lambda i: (0, 0)),
            pl.BlockSpec((1, 4), lambda i: (0, 0)),
        ],
        out_specs=[pl.BlockSpec((4, _BN), lambda i: (0, i))],
        out_shape=[jax.ShapeDtypeStruct((4, N_ACC), jnp.float32)],
    )(x_t, xcp, agg, w1a, w1b, w1c, b1, w2, b2, w3, b3)[0]


def kernel(x, edge_index, edge_attr, h_node_h, h_node_c, h_edge_h, h_edge_c,
           Wih_n, Whh_n, bih_n, bhh_n, Wih_e, Whh_e, bih_e, bhh_e,
           We1, be1, We2, be2, Wc1, bc1, Wc2, bc2, Wc3, bc3):
    row = edge_index[0]
    col = edge_index[1]
    idx2 = jnp.pad(row, (0, E_PAD - E), constant_values=DUMMY).reshape(NCH, CH)
    idxg = jnp.pad(jnp.concatenate([row, col]), (0, G_PAD - 2 * E))

    ea_t = edge_attr.T
    b_e = (bih_e + bhh_e)[:, None]
    h_e2, c_e2, h2pad = _k1(ea_t, h_edge_h[0].T, h_edge_c[0].T,
                            Wih_e, Whh_e, b_e)

    p = _s1(h2pad, idx2)

    b_n = bih_n + bhh_n
    wi4 = [Wih_n[k * RNN:(k + 1) * RNN] for k in range(4)]
    wh4 = [Whh_n[k * RNN:(k + 1) * RNN] for k in range(4)]
    b4 = [b_n[k * RNN:(k + 1) * RNN][:, None] for k in range(4)]
    npad = ((0, 0), (0, N_ACC - N))
    x_t = jnp.pad(x.T, npad)
    h_n2, c_n2, xcp = _k2(x_t, jnp.pad(h_node_h[0].T, npad),
                          jnp.pad(h_node_c[0].T, npad), p, wi4, wh4, b4)

    g = _sg(xcp, idxg).reshape(G_PAD // 4, 128)

    eye4 = jnp.eye(4, dtype=jnp.float32)
    w1a = jnp.kron(eye4, jnp.pad(We1[:28], ((0, 4), (0, 0))))
    w1b = jnp.kron(eye4, jnp.pad(We1[28:56], ((0, 4), (0, 0))))
    w1c = jnp.kron(eye4, We1[56:60])
    w2 = jnp.kron(eye4, We2)
    eap = edge_attr.reshape(E // 4, 16)
    eo = _k3(g, eap, w1a, w1b, w1c, jnp.tile(be1, 4)[None, :],
             w2, jnp.tile(be2, 4)[None, :])

    agg = _s2(eo.reshape(E_PAD, 32), idx2)

    wc1a = jnp.pad(Wc1[:28], ((0, 4), (0, 0)))
    out_t = _k4(x_t, xcp, agg, wc1a, Wc1[28:44], Wc1[44:60], bc1[None, :],
                Wc2, bc2[None, :], Wc3, bc3[None, :])

    return (out_t[:, :N].T, h_n2[:, :N].T[None], c_n2[:, :N].T[None],
            h_e2.T[None], c_e2.T[None])
